# baseline (speedup 1.0000x reference)
import functools

import jax
import jax.numpy as jnp
from jax import lax
from jax.experimental import pallas as pl
from jax.experimental.pallas import tpu as pltpu
from jax.experimental.pallas import tpu_sc as plsc

N_USERS = 50000
N_NODES = 100000
D = 32
DH = 16
E = 1600000
LAYERS = 3

NC, NS = 2, 16
W = 128
K = 8
KE = 5
NP = 100096
EP = 1638400
ROWS_ALL = EP // W
RPT_MAIN = ROWS_ALL // NS
RPT_DEG = ROWS_ALL // (NC * NS)
NODE_RPT = NP // NS
ZB = NODE_RPT // 8
NR = NODE_RPT // 16

_mesh = plsc.VectorSubcoreMesh(core_axis_name="c", subcore_axis_name="s")
_sc_params = pltpu.CompilerParams(use_tc_tiling_on_sc=False)


def _fill(ref, val, n):
    def body(i, carry):
        ref[i] = jnp.full((DH,), val, jnp.float32)
        return carry
    lax.fori_loop(0, n, body, 0)


def _zero_acc(acc, zbuf, s, zb=ZB):
    for k in range(NODE_RPT // zb):
        pltpu.sync_copy(zbuf.at[pl.ds(0, zb)],
                        acc.at[pl.ds(s * NODE_RPT + k * zb, zb)])


def _copy_out(acc, zbuf, out_hbm, base, s, zb=ZB):
    for k in range(NODE_RPT // zb):
        off = s * NODE_RPT + k * zb
        pltpu.sync_copy(acc.at[pl.ds(off, zb)], zbuf.at[pl.ds(0, zb)])
        pltpu.sync_copy(zbuf.at[pl.ds(0, zb)], out_hbm.at[pl.ds(base + off, zb)])


@functools.partial(
    pl.kernel,
    out_type=jax.ShapeDtypeStruct((NC * NP, DH), jnp.float32),
    mesh=_mesh,
    scratch_types=[
        pltpu.VMEM((K, W), jnp.int32),
        pltpu.VMEM((W, DH), jnp.float32),
        pltpu.VMEM((ZB, DH), jnp.float32),
        pltpu.VMEM_SHARED((NP, DH), jnp.float32),
        pltpu.SemaphoreType.DMA,
    ],
    compiler_params=_sc_params,
)
def _deg_kernel(col2d, out, colv, ones, zbuf, acc, sem):
    c = lax.axis_index("c")
    s = lax.axis_index("s")
    _fill(zbuf, 0.0, ZB)
    _fill(ones, 1.0, W)
    _zero_acc(acc, zbuf, s)
    plsc.subcore_barrier()

    tile_base = (c * NS + s) * RPT_DEG

    pltpu.sync_copy(col2d.at[pl.ds(tile_base, K), :], colv)
    for j in range(K):
        pltpu.async_copy(zbuf.at[pl.ds(0, W)], acc.at[colv.at[j]], sem,
                         add=True)

    def chunk(m, carry):
        for j in range(K):
            pltpu.make_async_copy(
                out.at[pl.ds(0, W)], zbuf.at[pl.ds(0, W)], sem).wait()
        pltpu.sync_copy(col2d.at[pl.ds(tile_base + m * K, K), :], colv)
        for j in range(K):
            pltpu.async_copy(ones, acc.at[colv.at[j]], sem, add=True)
        return carry

    lax.fori_loop(0, RPT_DEG // K, chunk, 0)
    for j in range(K):
        pltpu.make_async_copy(
            out.at[pl.ds(0, W)], zbuf.at[pl.ds(0, W)], sem).wait()
    plsc.subcore_barrier()
    _copy_out(acc, zbuf, out, c * NP, s)


@functools.partial(
    pl.kernel,
    out_type=(
        jax.ShapeDtypeStruct((NC * NP, DH), jnp.float32),
        jax.ShapeDtypeStruct((NC * NP, DH), jnp.float32),
        jax.ShapeDtypeStruct((NC * NP, DH), jnp.float32),
        jax.ShapeDtypeStruct((NC * NP, DH), jnp.float32),
    ),
    mesh=_mesh,
    scratch_types=[
        pltpu.VMEM((2, KE, W), jnp.int32),
        pltpu.VMEM((4, KE, W), jnp.int32),
        pltpu.VMEM((2, KE * W, DH), jnp.float32),
        pltpu.VMEM_SHARED((NP, DH), jnp.float32),
        pltpu.SemaphoreType.DMA,
        pltpu.SemaphoreType.DMA,
        pltpu.SemaphoreType.DMA,
    ],
    compiler_params=_sc_params,
)
def _gcn_kernel(rowadj2d, col2d, y0, dinv2, out1, out2, out3, ybuf,
                rowv, colv, msg, acc, gsem, ssem, isem):
    c = lax.axis_index("c")
    s = lax.axis_index("s")

    tile_base = c * ROWS_ALL + s * RPT_MAIN
    col_base = s * RPT_MAIN
    last = RPT_MAIN // KE - 1

    def drain_scatters(n):
        for _ in range(n):
            pltpu.make_async_copy(
                y0.at[pl.ds(0, W)], msg.at[1, pl.ds(0, W)], ssem).wait()

    def wait_idx(b):
        for _ in range(2):
            pltpu.make_async_copy(
                rowadj2d.at[pl.ds(0, KE), :], rowv.at[b], isem).wait()

    def edge_pass(y):
        _fill(msg.at[0], 0.0, KE * W)
        _fill(msg.at[1], 0.0, KE * W)
        _zero_acc(acc, msg.at[0], s, NR)
        plsc.subcore_barrier()

        pltpu.sync_copy(col2d.at[pl.ds(col_base, KE), :], colv.at[3])
        for j in range(2 * KE):
            pltpu.async_copy(msg.at[1, pl.ds((j % KE) * W, W)],
                             acc.at[colv.at[3, j % KE]], ssem, add=True)
        pltpu.async_copy(rowadj2d.at[pl.ds(tile_base, KE), :], rowv.at[0],
                         isem)
        pltpu.async_copy(col2d.at[pl.ds(col_base, KE), :], colv.at[0], isem)
        pltpu.async_copy(col2d.at[pl.ds(col_base + KE, KE), :], colv.at[1],
                         isem)

        def section(b, q, m, m_pf_row, m_pf_col):
            drain_scatters(KE)
            wait_idx(b)
            gds = []
            for j in range(KE):
                gds.append(pltpu.async_copy(
                    y.at[rowv.at[b, j]], msg.at[b, pl.ds(j * W, W)], gsem))
            pltpu.async_copy(
                rowadj2d.at[pl.ds(tile_base + m_pf_row * KE, KE), :],
                rowv.at[1 - b], isem)
            pltpu.async_copy(
                col2d.at[pl.ds(col_base + m_pf_col * KE, KE), :],
                colv.at[(q + 2) % 4], isem)
            for j in range(KE):
                gds[j].wait()
                pltpu.async_copy(
                    msg.at[b, pl.ds(j * W, W)], acc.at[colv.at[q, j]], ssem,
                    add=True)

        def body(t, carry):
            m0 = 4 * t

            def cl(m):
                return jnp.minimum(m, last)

            section(0, 0, m0, cl(m0 + 1), cl(m0 + 2))
            section(1, 1, m0 + 1, cl(m0 + 2), cl(m0 + 3))
            section(0, 2, m0 + 2, cl(m0 + 3), cl(m0 + 4))
            section(1, 3, m0 + 3, cl(m0 + 4), cl(m0 + 5))
            return carry

        lax.fori_loop(0, RPT_MAIN // (4 * KE), body, 0)
        drain_scatters(2 * KE)
        for _ in range(3):
            pltpu.make_async_copy(
                rowadj2d.at[pl.ds(0, KE), :], rowv.at[0], isem).wait()
        plsc.subcore_barrier()

    def node_pass(out_hbm, write_y):
        for k in range(NODE_RPT // NR):
            off = s * NODE_RPT + k * NR
            pltpu.sync_copy(acc.at[pl.ds(off, NR)], msg.at[0, pl.ds(0, NR)])
            pltpu.sync_copy(msg.at[0, pl.ds(0, NR)],
                            out_hbm.at[pl.ds(c * NP + off, NR)])
            if write_y:
                pltpu.sync_copy(dinv2.at[pl.ds(off, NR)],
                                msg.at[1, pl.ds(0, NR)])

                def mul(r, carry):
                    msg[0, r] = msg[0, r] * msg[1, r]
                    return carry

                lax.fori_loop(0, NR, mul, 0)
                pltpu.sync_copy(msg.at[0, pl.ds(0, NR)],
                                ybuf.at[pl.ds(c * NP + off, NR)])
        plsc.subcore_barrier()

    edge_pass(y0)
    node_pass(out1, True)
    edge_pass(ybuf)
    node_pass(out2, True)
    edge_pass(ybuf)
    node_pass(out3, False)


_TCROWS = 3128


def _prep_body(emb_ref, dega_ref, degb_ref, dinv_ref, dinv2_ref, y0_ref):
    c = pl.program_id(0)
    deg = dega_ref[...] + degb_ref[...]
    dinv = jnp.where(deg > 0, lax.rsqrt(deg), 0.0)
    dinv_ref[...] = dinv
    dinv2_ref[...] = dinv * dinv
    e = emb_ref[...]
    half = jnp.where(c == 0, e[:, :DH], e[:, DH:])
    y0_ref[...] = half * dinv


def _tc_prep(emb_p, deg_flat):
    nb = NP // _TCROWS
    return pl.pallas_call(
        _prep_body,
        grid=(NC, nb),
        in_specs=[
            pl.BlockSpec((_TCROWS, D), lambda c, i: (i, 0)),
            pl.BlockSpec((_TCROWS, DH), lambda c, i: (i, 0)),
            pl.BlockSpec((_TCROWS, DH), lambda c, i: (i + NP // _TCROWS, 0)),
        ],
        out_specs=[
            pl.BlockSpec((_TCROWS, DH), lambda c, i: (i, 0)),
            pl.BlockSpec((_TCROWS, DH), lambda c, i: (i, 0)),
            pl.BlockSpec((_TCROWS, DH),
                         lambda c, i: (c * (NP // _TCROWS) + i, 0)),
        ],
        out_shape=[
            jax.ShapeDtypeStruct((NP, DH), jnp.float32),
            jax.ShapeDtypeStruct((NP, DH), jnp.float32),
            jax.ShapeDtypeStruct((NC * NP, DH), jnp.float32),
        ],
    )(emb_p, deg_flat, deg_flat)


def _final_body(emb_ref, a10, a11, a20, a21, a30, a31, dinv_ref, out_ref):
    dinv = dinv_ref[...]
    s0 = a10[...] + a20[...] + a30[...]
    s1 = a11[...] + a21[...] + a31[...]
    e = emb_ref[...]
    out_ref[:, :DH] = (e[:, :DH] + dinv * s0) * 0.25
    out_ref[:, DH:] = (e[:, DH:] + dinv * s1) * 0.25


def _tc_final(emb_p, a1, a2, a3, dinv_rep):
    nb = NP // _TCROWS
    lo = pl.BlockSpec((_TCROWS, DH), lambda i: (i, 0))
    hi = pl.BlockSpec((_TCROWS, DH), lambda i: (i + NP // _TCROWS, 0))
    return pl.pallas_call(
        _final_body,
        grid=(nb,),
        in_specs=[pl.BlockSpec((_TCROWS, D), lambda i: (i, 0)),
                  lo, hi, lo, hi, lo, hi, lo],
        out_specs=pl.BlockSpec((_TCROWS, D), lambda i: (i, 0)),
        out_shape=jax.ShapeDtypeStruct((NP, D), jnp.float32),
    )(emb_p, a1, a1, a2, a2, a3, a3, dinv_rep)


def kernel(emb, edge_index):
    row = edge_index[0]
    col = edge_index[1]
    pad = EP - E
    row_p = jnp.concatenate([row, jnp.full((pad,), N_NODES, jnp.int32)])
    col_p = jnp.concatenate([col, jnp.full((pad,), N_NODES, jnp.int32)])
    rowadj2d = jnp.concatenate([row_p, row_p + NP]).reshape(2 * ROWS_ALL, W)
    col2d = col_p.reshape(ROWS_ALL, W)
    emb_p = jnp.pad(emb, ((0, NP - N_NODES), (0, 0)))

    deg_flat = _deg_kernel(col2d)
    dinv_rep, dinv2_rep, y0 = _tc_prep(emb_p, deg_flat)
    a1, a2, a3, _ = _gcn_kernel(rowadj2d, col2d, y0, dinv2_rep)
    out_full = _tc_final(emb_p, a1, a2, a3, dinv_rep)
    return (out_full[:N_USERS], out_full[N_USERS:N_NODES])


# speedup vs baseline: 1.0332x; 1.0332x over previous
import functools

import jax
import jax.numpy as jnp
from jax import lax
from jax.experimental import pallas as pl
from jax.experimental.pallas import tpu as pltpu
from jax.experimental.pallas import tpu_sc as plsc

N_USERS = 50000
N_NODES = 100000
D = 32
DH = 16
E = 1600000
LAYERS = 3

NC, NS = 2, 16
W = 128
K = 8
KE = 5
NP = 100096
EP = 1638400
ROWS_ALL = EP // W
RPT_MAIN = ROWS_ALL // NS
RPT_DEG = ROWS_ALL // (NC * NS)
NODE_RPT = NP // NS
ZB = NODE_RPT // 8
NR = NODE_RPT // 16

_mesh = plsc.VectorSubcoreMesh(core_axis_name="c", subcore_axis_name="s")
_sc_params = pltpu.CompilerParams(use_tc_tiling_on_sc=False)


def _fill(ref, val, n):
    def body(i, carry):
        ref[i] = jnp.full((DH,), val, jnp.float32)
        return carry
    lax.fori_loop(0, n, body, 0)


def _zero_acc(acc, zbuf, s, zb=ZB):
    for k in range(NODE_RPT // zb):
        pltpu.sync_copy(zbuf.at[pl.ds(0, zb)],
                        acc.at[pl.ds(s * NODE_RPT + k * zb, zb)])


def _copy_out(acc, zbuf, out_hbm, base, s, zb=ZB):
    for k in range(NODE_RPT // zb):
        off = s * NODE_RPT + k * zb
        pltpu.sync_copy(acc.at[pl.ds(off, zb)], zbuf.at[pl.ds(0, zb)])
        pltpu.sync_copy(zbuf.at[pl.ds(0, zb)], out_hbm.at[pl.ds(base + off, zb)])


@functools.partial(
    pl.kernel,
    out_type=jax.ShapeDtypeStruct((NC * NP, DH), jnp.float32),
    mesh=_mesh,
    scratch_types=[
        pltpu.VMEM((K, W), jnp.int32),
        pltpu.VMEM((W, DH), jnp.float32),
        pltpu.VMEM((ZB, DH), jnp.float32),
        pltpu.VMEM_SHARED((NP, DH), jnp.float32),
        pltpu.SemaphoreType.DMA,
    ],
    compiler_params=_sc_params,
)
def _deg_kernel(col2d, out, colv, ones, zbuf, acc, sem):
    c = lax.axis_index("c")
    s = lax.axis_index("s")
    _fill(zbuf, 0.0, ZB)
    _fill(ones, 1.0, W)
    _zero_acc(acc, zbuf, s)
    plsc.subcore_barrier()

    tile_base = (c * NS + s) * RPT_DEG

    pltpu.sync_copy(col2d.at[pl.ds(tile_base, K), :], colv)
    for j in range(K):
        pltpu.async_copy(zbuf.at[pl.ds(0, W)], acc.at[colv.at[j]], sem,
                         add=True)

    def chunk(m, carry):
        for j in range(K):
            pltpu.make_async_copy(
                out.at[pl.ds(0, W)], zbuf.at[pl.ds(0, W)], sem).wait()
        pltpu.sync_copy(col2d.at[pl.ds(tile_base + m * K, K), :], colv)
        for j in range(K):
            pltpu.async_copy(ones, acc.at[colv.at[j]], sem, add=True)
        return carry

    lax.fori_loop(0, RPT_DEG // K, chunk, 0)
    for j in range(K):
        pltpu.make_async_copy(
            out.at[pl.ds(0, W)], zbuf.at[pl.ds(0, W)], sem).wait()
    plsc.subcore_barrier()
    _copy_out(acc, zbuf, out, c * NP, s)


@functools.partial(
    pl.kernel,
    out_type=(
        jax.ShapeDtypeStruct((NC, NP, DH), jnp.float32),
        jax.ShapeDtypeStruct((NC, NP, DH), jnp.float32),
        jax.ShapeDtypeStruct((NC, NP, DH), jnp.float32),
        jax.ShapeDtypeStruct((NC * NP, DH), jnp.float32),
    ),
    mesh=_mesh,
    scratch_types=[
        pltpu.VMEM((2, KE, W), jnp.int32),
        pltpu.VMEM((4, KE, W), jnp.int32),
        pltpu.VMEM((2, KE * W, DH), jnp.float32),
        pltpu.VMEM_SHARED((NP, DH), jnp.float32),
        pltpu.SemaphoreType.DMA,
        pltpu.SemaphoreType.DMA,
        pltpu.SemaphoreType.DMA,
    ],
    compiler_params=_sc_params,
)
def _gcn_kernel(rowadj2d, col2d, y0, dinv2, out1, out2, out3, ybuf,
                rowv, colv, msg, acc, gsem, ssem, isem):
    c = lax.axis_index("c")
    s = lax.axis_index("s")

    tile_base = c * ROWS_ALL + s * RPT_MAIN
    col_base = s * RPT_MAIN
    last = RPT_MAIN // KE - 1

    def drain_scatters(n):
        for _ in range(n):
            pltpu.make_async_copy(
                y0.at[pl.ds(0, W)], msg.at[1, pl.ds(0, W)], ssem).wait()

    def wait_idx(b):
        for _ in range(2):
            pltpu.make_async_copy(
                rowadj2d.at[pl.ds(0, KE), :], rowv.at[b], isem).wait()

    def edge_pass(y):
        _fill(msg.at[0], 0.0, KE * W)
        _fill(msg.at[1], 0.0, KE * W)
        _zero_acc(acc, msg.at[0], s, NR)
        plsc.subcore_barrier()

        pltpu.sync_copy(col2d.at[pl.ds(col_base, KE), :], colv.at[3])
        for j in range(2 * KE):
            pltpu.async_copy(msg.at[1, pl.ds((j % KE) * W, W)],
                             acc.at[colv.at[3, j % KE]], ssem, add=True)
        pltpu.async_copy(rowadj2d.at[pl.ds(tile_base, KE), :], rowv.at[0],
                         isem)
        pltpu.async_copy(col2d.at[pl.ds(col_base, KE), :], colv.at[0], isem)
        pltpu.async_copy(col2d.at[pl.ds(col_base + KE, KE), :], colv.at[1],
                         isem)

        def section(b, q, m, m_pf_row, m_pf_col):
            drain_scatters(KE)
            wait_idx(b)
            gds = []
            for j in range(KE):
                gds.append(pltpu.async_copy(
                    y.at[rowv.at[b, j]], msg.at[b, pl.ds(j * W, W)], gsem))
            pltpu.async_copy(
                rowadj2d.at[pl.ds(tile_base + m_pf_row * KE, KE), :],
                rowv.at[1 - b], isem)
            pltpu.async_copy(
                col2d.at[pl.ds(col_base + m_pf_col * KE, KE), :],
                colv.at[(q + 2) % 4], isem)
            for j in range(KE):
                gds[j].wait()
                pltpu.async_copy(
                    msg.at[b, pl.ds(j * W, W)], acc.at[colv.at[q, j]], ssem,
                    add=True)

        def body(t, carry):
            m0 = 4 * t

            def cl(m):
                return jnp.minimum(m, last)

            section(0, 0, m0, cl(m0 + 1), cl(m0 + 2))
            section(1, 1, m0 + 1, cl(m0 + 2), cl(m0 + 3))
            section(0, 2, m0 + 2, cl(m0 + 3), cl(m0 + 4))
            section(1, 3, m0 + 3, cl(m0 + 4), cl(m0 + 5))
            return carry

        lax.fori_loop(0, RPT_MAIN // (4 * KE), body, 0)
        drain_scatters(2 * KE)
        for _ in range(3):
            pltpu.make_async_copy(
                rowadj2d.at[pl.ds(0, KE), :], rowv.at[0], isem).wait()
        plsc.subcore_barrier()

    def node_pass(out_hbm, write_y):
        for k in range(NODE_RPT // NR):
            off = s * NODE_RPT + k * NR
            pltpu.sync_copy(acc.at[pl.ds(off, NR)], msg.at[0, pl.ds(0, NR)])
            pltpu.sync_copy(msg.at[0, pl.ds(0, NR)],
                            out_hbm.at[c, pl.ds(off, NR)])
            if write_y:
                pltpu.sync_copy(dinv2.at[pl.ds(off, NR)],
                                msg.at[1, pl.ds(0, NR)])

                def mul(r, carry):
                    msg[0, r] = msg[0, r] * msg[1, r]
                    return carry

                lax.fori_loop(0, NR, mul, 0)
                pltpu.sync_copy(msg.at[0, pl.ds(0, NR)],
                                ybuf.at[pl.ds(c * NP + off, NR)])
        plsc.subcore_barrier()

    edge_pass(y0)
    node_pass(out1, True)
    edge_pass(ybuf)
    node_pass(out2, True)
    edge_pass(ybuf)
    node_pass(out3, False)


_TCROWS = 3128


def _prep_body(emb_ref, dega_ref, degb_ref, dinv_ref, dinv2_ref, y0_ref):
    c = pl.program_id(0)
    deg = dega_ref[...] + degb_ref[...]
    dinv = jnp.where(deg > 0, lax.rsqrt(deg), 0.0)
    dinv_ref[...] = dinv
    dinv2_ref[...] = dinv * dinv
    e = emb_ref[...]
    half = jnp.where(c == 0, e[:, :DH], e[:, DH:])
    y0_ref[...] = half * dinv


def _tc_prep(emb_p, deg_flat):
    nb = NP // _TCROWS
    return pl.pallas_call(
        _prep_body,
        grid=(NC, nb),
        in_specs=[
            pl.BlockSpec((_TCROWS, D), lambda c, i: (i, 0)),
            pl.BlockSpec((_TCROWS, DH), lambda c, i: (i, 0)),
            pl.BlockSpec((_TCROWS, DH), lambda c, i: (i + NP // _TCROWS, 0)),
        ],
        out_specs=[
            pl.BlockSpec((_TCROWS, DH), lambda c, i: (i, 0)),
            pl.BlockSpec((_TCROWS, DH), lambda c, i: (i, 0)),
            pl.BlockSpec((_TCROWS, DH),
                         lambda c, i: (c * (NP // _TCROWS) + i, 0)),
        ],
        out_shape=[
            jax.ShapeDtypeStruct((NP, DH), jnp.float32),
            jax.ShapeDtypeStruct((NP, DH), jnp.float32),
            jax.ShapeDtypeStruct((NC * NP, DH), jnp.float32),
        ],
    )(emb_p, deg_flat, deg_flat)


def _final_body(emb_ref, a10, a11, a20, a21, a30, a31, dinv_ref, out_ref):
    dinv = dinv_ref[...]
    s0 = a10[0] + a20[0] + a30[0]
    s1 = a11[0] + a21[0] + a31[0]
    e = emb_ref[...]
    out_ref[:, :DH] = (e[:, :DH] + dinv * s0) * 0.25
    out_ref[:, DH:] = (e[:, DH:] + dinv * s1) * 0.25


_TCF = 5000


def _tc_final_half(emb, a1, a2, a3, dinv_rep, half):
    nb = N_USERS // _TCF
    base = half * nb

    def lo(i):
        return (0, base + i, 0)

    def hi(i):
        return (1, base + i, 0)

    a_lo = pl.BlockSpec((1, _TCF, DH), lo)
    a_hi = pl.BlockSpec((1, _TCF, DH), hi)
    return pl.pallas_call(
        _final_body,
        grid=(nb,),
        in_specs=[pl.BlockSpec((_TCF, D), lambda i: (base + i, 0)),
                  a_lo, a_hi, a_lo, a_hi, a_lo, a_hi,
                  pl.BlockSpec((_TCF, DH), lambda i: (base + i, 0))],
        out_specs=pl.BlockSpec((_TCF, D), lambda i: (i, 0)),
        out_shape=jax.ShapeDtypeStruct((N_NODES - N_USERS if half else N_USERS,
                                        D), jnp.float32),
    )(emb, a1, a1, a2, a2, a3, a3, dinv_rep)


def kernel(emb, edge_index):
    row = edge_index[0]
    col = edge_index[1]
    pad = EP - E
    row_p = jnp.concatenate([row, jnp.full((pad,), N_NODES, jnp.int32)])
    col_p = jnp.concatenate([col, jnp.full((pad,), N_NODES, jnp.int32)])
    rowadj2d = jnp.concatenate([row_p, row_p + NP]).reshape(2 * ROWS_ALL, W)
    col2d = col_p.reshape(ROWS_ALL, W)
    emb_p = jnp.pad(emb, ((0, NP - N_NODES), (0, 0)))

    deg_flat = _deg_kernel(col2d)
    dinv_rep, dinv2_rep, y0 = _tc_prep(emb_p, deg_flat)
    a1, a2, a3, _ = _gcn_kernel(rowadj2d, col2d, y0, dinv2_rep)
    users = _tc_final_half(emb, a1, a2, a3, dinv_rep, 0)
    items = _tc_final_half(emb, a1, a2, a3, dinv_rep, 1)
    return (users, items)
